# baseline (speedup 1.0000x reference)
import jax
import jax.numpy as jnp
from jax import lax
from jax.experimental import pallas as pl
from jax.experimental.pallas import tpu as pltpu
from jax.experimental.pallas import tpu_sc as plsc

N_NODES = 10000
IN_CH = 128
HID = 16
OUT_CH = 128
N_EDGES = 320000

NC = 2
NS = 16
CHUNK = 128
N_PAD = 10240
EPT = 10240
E_PAD = NC * NS * EPT
K_CHUNKS = EPT // CHUNK
RPT = N_PAD // NS
ROW_BLK = 400


def _make_agg(feat, with_gather):
    mesh = plsc.VectorSubcoreMesh(core_axis_name="c", subcore_axis_name="s")
    NB = 4
    KD = 16
    scratch = [
        pltpu.VMEM((K_CHUNKS, CHUNK), jnp.int32),
        pltpu.VMEM((K_CHUNKS, CHUNK), jnp.int32),
        pltpu.VMEM((NB, CHUNK, feat), jnp.float32),
        pltpu.VMEM_SHARED((N_PAD, feat), jnp.float32),
        pltpu.SemaphoreType.DMA((NB,)),
    ]

    def body(*refs):
        if with_gather:
            (t_hbm, src_hbm, dst_hbm, out_hbm,
             src_v, dst_v, rows_v, acc_sh, sems) = refs
        else:
            (src_hbm, dst_hbm, out_hbm,
             src_v, dst_v, rows_v, acc_sh, sems) = refs
        c = lax.axis_index("c")
        s = lax.axis_index("s")
        wid = s * NC + c

        pltpu.sync_copy(src_hbm.at[pl.ds(wid * K_CHUNKS, K_CHUNKS)], src_v)
        pltpu.sync_copy(dst_hbm.at[pl.ds(wid * K_CHUNKS, K_CHUNKS)], dst_v)

        def zstore(i, carry):
            rows_v[0, i, :] = jnp.zeros((feat,), jnp.float32)
            return carry
        lax.fori_loop(0, CHUNK, zstore, 0)
        for b in range(RPT // CHUNK):
            pltpu.sync_copy(
                rows_v.at[0], acc_sh.at[pl.ds(s * RPT + b * CHUNK, CHUNK)])

        if not with_gather:
            def ostore(i, carry):
                rows_v[0, i, :] = jnp.ones((feat,), jnp.float32)
                return carry
            lax.fori_loop(0, CHUNK, ostore, 0)

        plsc.subcore_barrier()

        if with_gather:
            for b in range(NB):
                pltpu.async_copy(
                    t_hbm.at[src_v.at[b]], rows_v.at[b], sems.at[b])

            def group(g, carry):
                for b in range(NB):
                    j = g * NB + b
                    pltpu.make_async_copy(
                        t_hbm.at[src_v.at[0]], rows_v.at[b],
                        sems.at[b]).wait()
                    pltpu.sync_copy(
                        rows_v.at[b], acc_sh.at[dst_v.at[j]], add=True)
                    nxt = j + NB

                    @pl.when(nxt < K_CHUNKS)
                    def _():
                        pltpu.async_copy(
                            t_hbm.at[src_v.at[nxt]], rows_v.at[b],
                            sems.at[b])
                return carry
            lax.fori_loop(0, K_CHUNKS // NB, group, 0)
        else:
            def dgroup(g, carry):
                def fire(j, carry2):
                    pltpu.async_copy(
                        rows_v.at[0], acc_sh.at[dst_v.at[g * KD + j]],
                        sems.at[0], add=True)
                    return carry2
                lax.fori_loop(0, KD, fire, 0)

                def drain(j, carry2):
                    pltpu.make_async_copy(
                        rows_v.at[0], acc_sh.at[dst_v.at[0]],
                        sems.at[0]).wait()
                    return carry2
                lax.fori_loop(0, KD, drain, 0)
                return carry
            lax.fori_loop(0, K_CHUNKS // KD, dgroup, 0)

        plsc.subcore_barrier()
        pltpu.sync_copy(acc_sh.at[pl.ds(s * RPT, RPT)],
                        out_hbm.at[c, pl.ds(s * RPT, RPT)])

    return pl.kernel(
        body,
        out_type=jax.ShapeDtypeStruct((NC, N_PAD, feat), jnp.float32),
        mesh=mesh,
        scratch_types=scratch,
        compiler_params=pltpu.CompilerParams(use_tc_tiling_on_sc=False),
    )


def _tc1_body(x_ref, w1_ref, dcnt_ref, t1_ref, dis_ref):
    h = jnp.dot(x_ref[...], w1_ref[...],
                preferred_element_type=jnp.float32,
                precision=lax.Precision.HIGHEST)
    deg = dcnt_ref[0] + dcnt_ref[1] + 1.0
    dis = lax.rsqrt(deg)
    dis_ref[...] = dis
    t1_ref[...] = h * dis


def _tc2_body(agg_ref, t1_ref, dis_ref, b1_ref, t2_ref):
    dis = dis_ref[...]
    ssum = agg_ref[0] + agg_ref[1] + t1_ref[...]
    out1 = jnp.maximum(dis * ssum + b1_ref[...], 0.0)
    t2_ref[...] = out1 * dis


def _tc3_body(agg_ref, t2_ref, dis_ref, w2_ref, b2_ref, o_ref):
    mid = dis_ref[...] * (agg_ref[0] + agg_ref[1] + t2_ref[...])
    o = jnp.dot(mid, w2_ref[...],
                preferred_element_type=jnp.float32,
                precision=lax.Precision.HIGHEST) + b2_ref[...]
    m = jnp.max(o, axis=1, keepdims=True)
    lse = jnp.log(jnp.sum(jnp.exp(o - m), axis=1, keepdims=True))
    o_ref[...] = o - m - lse


_GRID = (N_NODES // ROW_BLK,)


def _tc1(x_p, W1, dcnt):
    return pl.pallas_call(
        _tc1_body,
        grid=_GRID,
        in_specs=[
            pl.BlockSpec((ROW_BLK, IN_CH), lambda i: (i, 0)),
            pl.BlockSpec((IN_CH, HID), lambda i: (0, 0)),
            pl.BlockSpec((NC, ROW_BLK, HID), lambda i: (0, i, 0)),
        ],
        out_specs=[
            pl.BlockSpec((ROW_BLK, HID), lambda i: (i, 0)),
            pl.BlockSpec((ROW_BLK, HID), lambda i: (i, 0)),
        ],
        out_shape=[
            jax.ShapeDtypeStruct((N_PAD, HID), jnp.float32),
            jax.ShapeDtypeStruct((N_PAD, HID), jnp.float32),
        ],
    )(x_p, W1, dcnt)


def _tc2(agg1, t1, dis, b1):
    return pl.pallas_call(
        _tc2_body,
        grid=_GRID,
        in_specs=[
            pl.BlockSpec((NC, ROW_BLK, HID), lambda i: (0, i, 0)),
            pl.BlockSpec((ROW_BLK, HID), lambda i: (i, 0)),
            pl.BlockSpec((ROW_BLK, HID), lambda i: (i, 0)),
            pl.BlockSpec((1, HID), lambda i: (0, 0)),
        ],
        out_specs=pl.BlockSpec((ROW_BLK, HID), lambda i: (i, 0)),
        out_shape=jax.ShapeDtypeStruct((N_PAD, HID), jnp.float32),
    )(agg1, t1, dis, b1)


def _tc3(agg2, t2, dis, W2, b2):
    return pl.pallas_call(
        _tc3_body,
        grid=_GRID,
        in_specs=[
            pl.BlockSpec((NC, ROW_BLK, HID), lambda i: (0, i, 0)),
            pl.BlockSpec((ROW_BLK, HID), lambda i: (i, 0)),
            pl.BlockSpec((ROW_BLK, HID), lambda i: (i, 0)),
            pl.BlockSpec((HID, OUT_CH), lambda i: (0, 0)),
            pl.BlockSpec((1, OUT_CH), lambda i: (0, 0)),
        ],
        out_specs=pl.BlockSpec((ROW_BLK, OUT_CH), lambda i: (i, 0)),
        out_shape=jax.ShapeDtypeStruct((N_NODES, OUT_CH), jnp.float32),
    )(agg2, t2, dis, W2, b2)


_sc_cache = {}


def _get_sc():
    if "agg" not in _sc_cache:
        _sc_cache["agg"] = _make_agg(HID, with_gather=True)
        _sc_cache["deg"] = _make_agg(HID, with_gather=False)
    return _sc_cache["agg"], _sc_cache["deg"]


@jax.jit
def kernel(x, edge_index, W1, b1, W2, b2):
    _agg16, _deg16 = _get_sc()
    src = edge_index[0].astype(jnp.int32)
    dst = edge_index[1].astype(jnp.int32)
    pad = E_PAD - N_EDGES
    fill = jnp.full((pad,), N_NODES, jnp.int32)
    src_p = jnp.concatenate([src, fill]).reshape(E_PAD // CHUNK, CHUNK)
    dst_p = jnp.concatenate([dst, fill]).reshape(E_PAD // CHUNK, CHUNK)

    dcnt = _deg16(src_p, dst_p)
    t1, dis = _tc1(x, W1, dcnt)
    agg1 = _agg16(t1, src_p, dst_p)
    t2 = _tc2(agg1, t1, dis, b1.reshape(1, HID))
    agg2 = _agg16(t2, src_p, dst_p)
    return _tc3(agg2, t2, dis, W2, b2.reshape(1, OUT_CH))


# speedup vs baseline: 1.2618x; 1.2618x over previous
import jax
import jax.numpy as jnp
from jax import lax
from jax.experimental import pallas as pl
from jax.experimental.pallas import tpu as pltpu
from jax.experimental.pallas import tpu_sc as plsc

N_NODES = 10000
IN_CH = 128
HID = 16
OUT_CH = 128
N_EDGES = 320000

NC = 2
NS = 16
CHUNK = 128
N_PAD = 10240
EPT = 10240
E_PAD = NC * NS * EPT
K_CHUNKS = EPT // CHUNK
RPT = N_PAD // NS
NKP = N_NODES // 8
NPK = N_PAD // 8


def _make_agg(feat, with_gather):
    mesh = plsc.VectorSubcoreMesh(core_axis_name="c", subcore_axis_name="s")
    NB = 4
    KD = 16
    scratch = [
        pltpu.VMEM((K_CHUNKS, CHUNK), jnp.int32),
        pltpu.VMEM((K_CHUNKS, CHUNK), jnp.int32),
        pltpu.VMEM((NB, CHUNK, feat), jnp.float32),
        pltpu.VMEM_SHARED((N_PAD, feat), jnp.float32),
        pltpu.SemaphoreType.DMA((NB,)),
    ]

    def body(*refs):
        if with_gather:
            (t_hbm, src_hbm, dst_hbm, out_hbm,
             src_v, dst_v, rows_v, acc_sh, sems) = refs
        else:
            (src_hbm, dst_hbm, out_hbm,
             src_v, dst_v, rows_v, acc_sh, sems) = refs
        c = lax.axis_index("c")
        s = lax.axis_index("s")
        wid = s * NC + c

        pltpu.sync_copy(src_hbm.at[pl.ds(wid * K_CHUNKS, K_CHUNKS)], src_v)
        pltpu.sync_copy(dst_hbm.at[pl.ds(wid * K_CHUNKS, K_CHUNKS)], dst_v)

        def zstore(i, carry):
            rows_v[0, i, :] = jnp.zeros((feat,), jnp.float32)
            return carry
        lax.fori_loop(0, CHUNK, zstore, 0)
        for b in range(RPT // CHUNK):
            pltpu.sync_copy(
                rows_v.at[0], acc_sh.at[pl.ds(s * RPT + b * CHUNK, CHUNK)])

        if not with_gather:
            def ostore(i, carry):
                rows_v[0, i, :] = jnp.ones((feat,), jnp.float32)
                return carry
            lax.fori_loop(0, CHUNK, ostore, 0)

        plsc.subcore_barrier()

        if with_gather:
            for b in range(NB):
                pltpu.async_copy(
                    t_hbm.at[src_v.at[b]], rows_v.at[b], sems.at[b])

            def group(g, carry):
                for b in range(NB):
                    j = g * NB + b
                    pltpu.make_async_copy(
                        t_hbm.at[src_v.at[0]], rows_v.at[b],
                        sems.at[b]).wait()
                    pltpu.sync_copy(
                        rows_v.at[b], acc_sh.at[dst_v.at[j]], add=True)
                    nxt = j + NB

                    @pl.when(nxt < K_CHUNKS)
                    def _():
                        pltpu.async_copy(
                            t_hbm.at[src_v.at[nxt]], rows_v.at[b],
                            sems.at[b])
                return carry
            lax.fori_loop(0, K_CHUNKS // NB, group, 0)
        else:
            def dgroup(g, carry):
                def fire(j, carry2):
                    pltpu.async_copy(
                        rows_v.at[0], acc_sh.at[dst_v.at[g * KD + j]],
                        sems.at[0], add=True)
                    return carry2
                lax.fori_loop(0, KD, fire, 0)

                def drain(j, carry2):
                    pltpu.make_async_copy(
                        rows_v.at[0], acc_sh.at[dst_v.at[0]],
                        sems.at[0]).wait()
                    return carry2
                lax.fori_loop(0, KD, drain, 0)
                return carry
            lax.fori_loop(0, K_CHUNKS // KD, dgroup, 0)

        plsc.subcore_barrier()
        pltpu.sync_copy(acc_sh.at[pl.ds(s * RPT, RPT)],
                        out_hbm.at[c, pl.ds(s * RPT, RPT)])

    return pl.kernel(
        body,
        out_type=jax.ShapeDtypeStruct((NC, N_PAD, feat), jnp.float32),
        mesh=mesh,
        scratch_types=scratch,
        compiler_params=pltpu.CompilerParams(use_tc_tiling_on_sc=False),
    )


def _tc1_body(x_ref, w1_ref, dcnt_ref, t1_ref, dis_ref):
    h = jnp.dot(x_ref[...], w1_ref[...],
                preferred_element_type=jnp.float32,
                precision=lax.Precision.HIGHEST)
    hpad = jnp.concatenate(
        [h, jnp.zeros((N_PAD - N_NODES, HID), jnp.float32)], 0)
    hp = jnp.concatenate(
        [hpad[i * NPK:(i + 1) * NPK] for i in range(8)], axis=1)
    deg = dcnt_ref[0] + dcnt_ref[1] + 1.0
    dis = lax.rsqrt(deg)
    dis_ref[...] = dis
    t1_ref[...] = hp * dis


def _tc2_body(agg_ref, t1_ref, dis_ref, b1_ref, t2_ref):
    dis = dis_ref[...]
    ssum = agg_ref[0] + agg_ref[1] + t1_ref[...]
    out1 = jnp.maximum(dis * ssum + b1_ref[...], 0.0)
    t2_ref[...] = out1 * dis


def _tc3_body(agg_ref, t2_ref, dis_ref, w2_ref, b2_ref, o_ref):
    midp = dis_ref[...] * (agg_ref[0] + agg_ref[1] + t2_ref[...])
    mid = jnp.concatenate(
        [midp[:, HID * i:HID * (i + 1)] for i in range(8)], axis=0)[:N_NODES]
    o = jnp.dot(mid, w2_ref[...],
                preferred_element_type=jnp.float32,
                precision=lax.Precision.HIGHEST) + b2_ref[...]
    m = jnp.max(o, axis=1, keepdims=True)
    lse = jnp.log(jnp.sum(jnp.exp(o - m), axis=1, keepdims=True))
    o_ref[...] = o - m - lse


def _tc1(x, W1, dcnt_p):
    return pl.pallas_call(
        _tc1_body,
        grid=(1,),
        in_specs=[
            pl.BlockSpec((N_NODES, IN_CH), lambda i: (0, 0)),
            pl.BlockSpec((IN_CH, HID), lambda i: (0, 0)),
            pl.BlockSpec((NC, NPK, 128), lambda i: (0, 0, 0)),
        ],
        out_specs=[
            pl.BlockSpec((NPK, 128), lambda i: (0, 0)),
            pl.BlockSpec((NPK, 128), lambda i: (0, 0)),
        ],
        out_shape=[
            jax.ShapeDtypeStruct((NPK, 128), jnp.float32),
            jax.ShapeDtypeStruct((NPK, 128), jnp.float32),
        ],
    )(x, W1, dcnt_p)


def _tc2(agg1_p, t1_p, dis_p, b1t):
    return pl.pallas_call(
        _tc2_body,
        grid=(1,),
        in_specs=[
            pl.BlockSpec((NC, NPK, 128), lambda i: (0, 0, 0)),
            pl.BlockSpec((NPK, 128), lambda i: (0, 0)),
            pl.BlockSpec((NPK, 128), lambda i: (0, 0)),
            pl.BlockSpec((1, 128), lambda i: (0, 0)),
        ],
        out_specs=pl.BlockSpec((NPK, 128), lambda i: (0, 0)),
        out_shape=jax.ShapeDtypeStruct((NPK, 128), jnp.float32),
    )(agg1_p, t1_p, dis_p, b1t)


def _tc3(agg2_p, t2_p, dis_p, W2, b2):
    return pl.pallas_call(
        _tc3_body,
        grid=(1,),
        in_specs=[
            pl.BlockSpec((NC, NPK, 128), lambda i: (0, 0, 0)),
            pl.BlockSpec((NPK, 128), lambda i: (0, 0)),
            pl.BlockSpec((NPK, 128), lambda i: (0, 0)),
            pl.BlockSpec((HID, OUT_CH), lambda i: (0, 0)),
            pl.BlockSpec((1, OUT_CH), lambda i: (0, 0)),
        ],
        out_specs=pl.BlockSpec((N_NODES, OUT_CH), lambda i: (0, 0)),
        out_shape=jax.ShapeDtypeStruct((N_NODES, OUT_CH), jnp.float32),
    )(agg2_p, t2_p, dis_p, W2, b2)


_sc_cache = {}


def _get_sc():
    if "agg" not in _sc_cache:
        _sc_cache["agg"] = _make_agg(HID, with_gather=True)
        _sc_cache["deg"] = _make_agg(HID, with_gather=False)
    return _sc_cache["agg"], _sc_cache["deg"]


@jax.jit
def kernel(x, edge_index, W1, b1, W2, b2):
    _agg16, _deg16 = _get_sc()
    src = edge_index[0].astype(jnp.int32)
    dst = edge_index[1].astype(jnp.int32)
    src = (src % NPK) * 8 + src // NPK
    dst = (dst % NPK) * 8 + dst // NPK
    pad = E_PAD - N_EDGES
    trash = (N_NODES % NPK) * 8 + N_NODES // NPK
    fill = jnp.full((pad,), trash, jnp.int32)
    src_p = jnp.concatenate([src, fill]).reshape(E_PAD // CHUNK, CHUNK)
    dst_p = jnp.concatenate([dst, fill]).reshape(E_PAD // CHUNK, CHUNK)

    dcnt = _deg16(src_p, dst_p)
    dcnt_p = dcnt.reshape(NC, NPK, 128)
    t1_p, dis_p = _tc1(x, W1, dcnt_p)
    t1 = t1_p.reshape(N_PAD, HID)
    agg1 = _agg16(t1, src_p, dst_p)
    t2_p = _tc2(agg1.reshape(NC, NPK, 128), t1_p, dis_p,
                jnp.tile(b1, 8).reshape(1, 128))
    t2 = t2_p.reshape(N_PAD, HID)
    agg2 = _agg16(t2, src_p, dst_p)
    return _tc3(agg2.reshape(NC, NPK, 128), t2_p, dis_p, W2,
                b2.reshape(1, OUT_CH))


# speedup vs baseline: 1.5422x; 1.2222x over previous
import jax
import jax.numpy as jnp
from jax import lax
from jax.experimental import pallas as pl
from jax.experimental.pallas import tpu as pltpu
from jax.experimental.pallas import tpu_sc as plsc

N_NODES = 10000
IN_CH = 128
HID = 16
OUT_CH = 128
N_EDGES = 320000

NC = 2
NS = 16
CHUNK = 128
N_PAD = 10240
EPT = 10240
E_PAD = NC * NS * EPT
K_CHUNKS = EPT // CHUNK
KC0 = 112
KC1 = 48
RPT = N_PAD // NS
NKP = N_NODES // 8
NPK = N_PAD // 8


def _make_agg(feat, with_gather):
    mesh = plsc.VectorSubcoreMesh(core_axis_name="c", subcore_axis_name="s")
    NB = 4
    KD = 16
    scratch = [
        pltpu.VMEM((KC0, CHUNK), jnp.int32),
        pltpu.VMEM((KC0, CHUNK), jnp.int32),
        pltpu.VMEM((NB, CHUNK, feat), jnp.float32),
        pltpu.VMEM_SHARED((N_PAD, feat), jnp.float32),
        pltpu.SemaphoreType.DMA((NB,)),
    ]

    def body(*refs):
        if with_gather:
            (t_hbm, src_hbm, dst_hbm, out_hbm,
             src_v, dst_v, rows_v, acc_sh, sems) = refs
        else:
            (src_hbm, dst_hbm, out_hbm,
             src_v, dst_v, rows_v, acc_sh, sems) = refs
        c = lax.axis_index("c")
        s = lax.axis_index("s")
        base = jnp.where(c == 0, s * KC0, NS * KC0 + s * KC1)
        nk = jnp.where(c == 0, KC0, KC1)

        @pl.when(c == 0)
        def _():
            pltpu.sync_copy(src_hbm.at[pl.ds(base, KC0)],
                            src_v.at[pl.ds(0, KC0)])
            pltpu.sync_copy(dst_hbm.at[pl.ds(base, KC0)],
                            dst_v.at[pl.ds(0, KC0)])

        @pl.when(c == 1)
        def _():
            pltpu.sync_copy(src_hbm.at[pl.ds(base, KC1)],
                            src_v.at[pl.ds(0, KC1)])
            pltpu.sync_copy(dst_hbm.at[pl.ds(base, KC1)],
                            dst_v.at[pl.ds(0, KC1)])

        def zstore(i, carry):
            rows_v[0, i, :] = jnp.zeros((feat,), jnp.float32)
            return carry
        lax.fori_loop(0, CHUNK, zstore, 0)
        for b in range(RPT // CHUNK):
            pltpu.sync_copy(
                rows_v.at[0], acc_sh.at[pl.ds(s * RPT + b * CHUNK, CHUNK)])

        if not with_gather:
            def ostore(i, carry):
                rows_v[0, i, :] = jnp.ones((feat,), jnp.float32)
                return carry
            lax.fori_loop(0, CHUNK, ostore, 0)

        plsc.subcore_barrier()

        if with_gather:
            for b in range(NB):
                pltpu.async_copy(
                    t_hbm.at[src_v.at[b]], rows_v.at[b], sems.at[b])

            def group(g, carry):
                for b in range(NB):
                    j = g * NB + b
                    pltpu.make_async_copy(
                        t_hbm.at[src_v.at[0]], rows_v.at[b],
                        sems.at[b]).wait()
                    pltpu.sync_copy(
                        rows_v.at[b], acc_sh.at[dst_v.at[j]], add=True)
                    nxt = j + NB

                    @pl.when(nxt < nk)
                    def _():
                        pltpu.async_copy(
                            t_hbm.at[src_v.at[nxt]], rows_v.at[b],
                            sems.at[b])
                return carry
            lax.fori_loop(0, nk // NB, group, 0)
        else:
            def dgroup(g, carry):
                def fire(j, carry2):
                    pltpu.async_copy(
                        rows_v.at[0], acc_sh.at[dst_v.at[g * KD + j]],
                        sems.at[0], add=True)
                    return carry2
                lax.fori_loop(0, KD, fire, 0)

                def drain(j, carry2):
                    pltpu.make_async_copy(
                        rows_v.at[0], acc_sh.at[dst_v.at[0]],
                        sems.at[0]).wait()
                    return carry2
                lax.fori_loop(0, KD, drain, 0)
                return carry
            lax.fori_loop(0, nk // KD, dgroup, 0)

        plsc.subcore_barrier()
        pltpu.sync_copy(acc_sh.at[pl.ds(s * RPT, RPT)],
                        out_hbm.at[c, pl.ds(s * RPT, RPT)])

    return pl.kernel(
        body,
        out_type=jax.ShapeDtypeStruct((NC, N_PAD, feat), jnp.float32),
        mesh=mesh,
        scratch_types=scratch,
        compiler_params=pltpu.CompilerParams(use_tc_tiling_on_sc=False),
    )


def _tc1_body(x_ref, w1_ref, dcnt_ref, t1_ref, dis_ref):
    h = jnp.dot(x_ref[...], w1_ref[...],
                preferred_element_type=jnp.float32,
                precision=lax.Precision.HIGHEST)
    hpad = jnp.concatenate(
        [h, jnp.zeros((N_PAD - N_NODES, HID), jnp.float32)], 0)
    hp = jnp.concatenate(
        [hpad[i * NPK:(i + 1) * NPK] for i in range(8)], axis=1)
    deg = dcnt_ref[0] + dcnt_ref[1] + 1.0
    dis = lax.rsqrt(deg)
    dis_ref[...] = dis
    t1_ref[...] = hp * dis


def _tc2_body(agg_ref, t1_ref, dis_ref, b1_ref, t2_ref):
    dis = dis_ref[...]
    ssum = agg_ref[0] + agg_ref[1] + t1_ref[...]
    out1 = jnp.maximum(dis * ssum + b1_ref[...], 0.0)
    t2_ref[...] = out1 * dis


def _tc3_body(agg_ref, t2_ref, dis_ref, w2_ref, b2_ref, o_ref):
    midp = dis_ref[...] * (agg_ref[0] + agg_ref[1] + t2_ref[...])
    mid = jnp.concatenate(
        [midp[:, HID * i:HID * (i + 1)] for i in range(8)], axis=0)[:N_NODES]
    o = jnp.dot(mid, w2_ref[...],
                preferred_element_type=jnp.float32,
                precision=lax.Precision.HIGHEST) + b2_ref[...]
    m = jnp.max(o, axis=1, keepdims=True)
    lse = jnp.log(jnp.sum(jnp.exp(o - m), axis=1, keepdims=True))
    o_ref[...] = o - m - lse


def _tc1(x, W1, dcnt_p):
    return pl.pallas_call(
        _tc1_body,
        grid=(1,),
        in_specs=[
            pl.BlockSpec((N_NODES, IN_CH), lambda i: (0, 0)),
            pl.BlockSpec((IN_CH, HID), lambda i: (0, 0)),
            pl.BlockSpec((NC, NPK, 128), lambda i: (0, 0, 0)),
        ],
        out_specs=[
            pl.BlockSpec((NPK, 128), lambda i: (0, 0)),
            pl.BlockSpec((NPK, 128), lambda i: (0, 0)),
        ],
        out_shape=[
            jax.ShapeDtypeStruct((NPK, 128), jnp.float32),
            jax.ShapeDtypeStruct((NPK, 128), jnp.float32),
        ],
    )(x, W1, dcnt_p)


def _tc2(agg1_p, t1_p, dis_p, b1t):
    return pl.pallas_call(
        _tc2_body,
        grid=(1,),
        in_specs=[
            pl.BlockSpec((NC, NPK, 128), lambda i: (0, 0, 0)),
            pl.BlockSpec((NPK, 128), lambda i: (0, 0)),
            pl.BlockSpec((NPK, 128), lambda i: (0, 0)),
            pl.BlockSpec((1, 128), lambda i: (0, 0)),
        ],
        out_specs=pl.BlockSpec((NPK, 128), lambda i: (0, 0)),
        out_shape=jax.ShapeDtypeStruct((NPK, 128), jnp.float32),
    )(agg1_p, t1_p, dis_p, b1t)


def _tc3(agg2_p, t2_p, dis_p, W2, b2):
    return pl.pallas_call(
        _tc3_body,
        grid=(1,),
        in_specs=[
            pl.BlockSpec((NC, NPK, 128), lambda i: (0, 0, 0)),
            pl.BlockSpec((NPK, 128), lambda i: (0, 0)),
            pl.BlockSpec((NPK, 128), lambda i: (0, 0)),
            pl.BlockSpec((HID, OUT_CH), lambda i: (0, 0)),
            pl.BlockSpec((1, OUT_CH), lambda i: (0, 0)),
        ],
        out_specs=pl.BlockSpec((N_NODES, OUT_CH), lambda i: (0, 0)),
        out_shape=jax.ShapeDtypeStruct((N_NODES, OUT_CH), jnp.float32),
    )(agg2_p, t2_p, dis_p, W2, b2)


_sc_cache = {}


def _get_sc():
    if "agg" not in _sc_cache:
        _sc_cache["agg"] = _make_agg(HID, with_gather=True)
        _sc_cache["deg"] = _make_agg(HID, with_gather=False)
    return _sc_cache["agg"], _sc_cache["deg"]


@jax.jit
def kernel(x, edge_index, W1, b1, W2, b2):
    _agg16, _deg16 = _get_sc()
    src = edge_index[0].astype(jnp.int32)
    dst = edge_index[1].astype(jnp.int32)
    def rho(n):
        q = jnp.floor((n.astype(jnp.float32) + 0.5) *
                      (1.0 / NPK)).astype(jnp.int32)
        return (n - q * NPK) * 8 + q
    src = rho(src)
    dst = rho(dst)
    pad = E_PAD - N_EDGES
    trash = (N_NODES % NPK) * 8 + N_NODES // NPK
    fill = jnp.full((pad,), trash, jnp.int32)
    src_p = jnp.concatenate([src, fill]).reshape(E_PAD // CHUNK, CHUNK)
    dst_p = jnp.concatenate([dst, fill]).reshape(E_PAD // CHUNK, CHUNK)

    dcnt = _deg16(src_p, dst_p)
    dcnt_p = dcnt.reshape(NC, NPK, 128)
    t1_p, dis_p = _tc1(x, W1, dcnt_p)
    t1 = t1_p.reshape(N_PAD, HID)
    agg1 = _agg16(t1, src_p, dst_p)
    t2_p = _tc2(agg1.reshape(NC, NPK, 128), t1_p, dis_p,
                jnp.tile(b1, 8).reshape(1, 128))
    t2 = t2_p.reshape(N_PAD, HID)
    agg2 = _agg16(t2, src_p, dst_p)
    return _tc3(agg2.reshape(NC, NPK, 128), t2_p, dis_p, W2,
                b2.reshape(1, OUT_CH))


# speedup vs baseline: 1.6087x; 1.0431x over previous
import jax
import jax.numpy as jnp
from jax import lax
from jax.experimental import pallas as pl
from jax.experimental.pallas import tpu as pltpu
from jax.experimental.pallas import tpu_sc as plsc

N_NODES = 10000
IN_CH = 128
HID = 16
OUT_CH = 128
N_EDGES = 320000

NC = 2
NS = 16
CHUNK = 128
N_PAD = 10240
EPT = 10240
E_PAD = NC * NS * EPT
K_CHUNKS = EPT // CHUNK
KC0 = 112
KC1 = 48
RPT = N_PAD // NS
NKP = N_NODES // 8
NPK = N_PAD // 8


def _make_agg(feat, with_gather, kc0=KC0, kc1=KC1):
    mesh = plsc.VectorSubcoreMesh(core_axis_name="c", subcore_axis_name="s")
    NB = 4
    KD = 16
    scratch = [
        pltpu.VMEM((kc0, CHUNK), jnp.int32),
        pltpu.VMEM((kc0, CHUNK), jnp.int32),
        pltpu.VMEM((NB, CHUNK, feat), jnp.float32),
        pltpu.VMEM_SHARED((N_PAD, feat), jnp.float32),
        pltpu.SemaphoreType.DMA((NB,)),
    ]

    def body(*refs):
        if with_gather:
            (t_hbm, src_hbm, dst_hbm, out_hbm,
             src_v, dst_v, rows_v, acc_sh, sems) = refs
        else:
            (src_hbm, dst_hbm, out_hbm,
             src_v, dst_v, rows_v, acc_sh, sems) = refs
        c = lax.axis_index("c")
        s = lax.axis_index("s")
        base = jnp.where(c == 0, s * kc0, NS * kc0 + s * kc1)
        nk = jnp.where(c == 0, kc0, kc1)

        @pl.when(c == 0)
        def _():
            pltpu.sync_copy(src_hbm.at[pl.ds(base, kc0)],
                            src_v.at[pl.ds(0, kc0)])
            pltpu.sync_copy(dst_hbm.at[pl.ds(base, kc0)],
                            dst_v.at[pl.ds(0, kc0)])

        @pl.when(c == 1)
        def _():
            pltpu.sync_copy(src_hbm.at[pl.ds(base, kc1)],
                            src_v.at[pl.ds(0, kc1)])
            pltpu.sync_copy(dst_hbm.at[pl.ds(base, kc1)],
                            dst_v.at[pl.ds(0, kc1)])

        def zstore(i, carry):
            rows_v[0, i, :] = jnp.zeros((feat,), jnp.float32)
            return carry
        lax.fori_loop(0, CHUNK, zstore, 0)
        for b in range(RPT // CHUNK):
            pltpu.sync_copy(
                rows_v.at[0], acc_sh.at[pl.ds(s * RPT + b * CHUNK, CHUNK)])

        if not with_gather:
            def ostore(i, carry):
                rows_v[0, i, :] = jnp.ones((feat,), jnp.float32)
                return carry
            lax.fori_loop(0, CHUNK, ostore, 0)

        plsc.subcore_barrier()

        if with_gather:
            for b in range(NB):
                pltpu.async_copy(
                    t_hbm.at[src_v.at[b]], rows_v.at[b], sems.at[b])

            def group(g, carry):
                for b in range(NB):
                    j = g * NB + b
                    pltpu.make_async_copy(
                        t_hbm.at[src_v.at[0]], rows_v.at[b],
                        sems.at[b]).wait()
                    pltpu.sync_copy(
                        rows_v.at[b], acc_sh.at[dst_v.at[j]], add=True)
                    nxt = j + NB

                    @pl.when(nxt < nk)
                    def _():
                        pltpu.async_copy(
                            t_hbm.at[src_v.at[nxt]], rows_v.at[b],
                            sems.at[b])
                return carry
            lax.fori_loop(0, nk // NB, group, 0)
        else:
            def dgroup(g, carry):
                def fire(j, carry2):
                    pltpu.async_copy(
                        rows_v.at[0], acc_sh.at[dst_v.at[g * KD + j]],
                        sems.at[0], add=True)
                    return carry2
                lax.fori_loop(0, KD, fire, 0)

                def drain(j, carry2):
                    pltpu.make_async_copy(
                        rows_v.at[0], acc_sh.at[dst_v.at[0]],
                        sems.at[0]).wait()
                    return carry2
                lax.fori_loop(0, KD, drain, 0)
                return carry
            lax.fori_loop(0, nk // KD, dgroup, 0)

        plsc.subcore_barrier()
        pltpu.sync_copy(acc_sh.at[pl.ds(s * RPT, RPT)],
                        out_hbm.at[c, pl.ds(s * RPT, RPT)])

    return pl.kernel(
        body,
        out_type=jax.ShapeDtypeStruct((NC, N_PAD, feat), jnp.float32),
        mesh=mesh,
        scratch_types=scratch,
        compiler_params=pltpu.CompilerParams(use_tc_tiling_on_sc=False),
    )


def _tc1a_body(x_ref, w1_ref, hp_ref):
    h = jnp.dot(x_ref[...], w1_ref[...],
                preferred_element_type=jnp.float32,
                precision=lax.Precision.HIGHEST)
    hpad = jnp.concatenate(
        [h, jnp.zeros((N_PAD - N_NODES, HID), jnp.float32)], 0)
    hp_ref[...] = jnp.concatenate(
        [hpad[i * NPK:(i + 1) * NPK] for i in range(8)], axis=1)


def _tc1b_body(hp_ref, dcnt_ref, t1_ref, dis_ref):
    deg = dcnt_ref[0] + dcnt_ref[1] + 1.0
    dis = lax.rsqrt(deg)
    dis_ref[...] = dis
    t1_ref[...] = hp_ref[...] * dis


def _tc2_body(agg_ref, t1_ref, dis_ref, b1_ref, t2_ref):
    dis = dis_ref[...]
    ssum = agg_ref[0] + agg_ref[1] + t1_ref[...]
    out1 = jnp.maximum(dis * ssum + b1_ref[...], 0.0)
    t2_ref[...] = out1 * dis


def _tc3_body(agg_ref, t2_ref, dis_ref, w2_ref, b2_ref, o_ref):
    midp = dis_ref[...] * (agg_ref[0] + agg_ref[1] + t2_ref[...])
    mid = jnp.concatenate(
        [midp[:, HID * i:HID * (i + 1)] for i in range(8)], axis=0)[:N_NODES]
    o = jnp.dot(mid, w2_ref[...],
                preferred_element_type=jnp.float32,
                precision=lax.Precision.HIGHEST) + b2_ref[...]
    m = jnp.max(o, axis=1, keepdims=True)
    lse = jnp.log(jnp.sum(jnp.exp(o - m), axis=1, keepdims=True))
    o_ref[...] = o - m - lse


def _tc1a(x, W1):
    return pl.pallas_call(
        _tc1a_body,
        grid=(1,),
        in_specs=[
            pl.BlockSpec((N_NODES, IN_CH), lambda i: (0, 0)),
            pl.BlockSpec((IN_CH, HID), lambda i: (0, 0)),
        ],
        out_specs=pl.BlockSpec((NPK, 128), lambda i: (0, 0)),
        out_shape=jax.ShapeDtypeStruct((NPK, 128), jnp.float32),
    )(x, W1)


def _tc1b(hp, dcnt_p):
    return pl.pallas_call(
        _tc1b_body,
        grid=(1,),
        in_specs=[
            pl.BlockSpec((NPK, 128), lambda i: (0, 0)),
            pl.BlockSpec((NC, NPK, 128), lambda i: (0, 0, 0)),
        ],
        out_specs=[
            pl.BlockSpec((NPK, 128), lambda i: (0, 0)),
            pl.BlockSpec((NPK, 128), lambda i: (0, 0)),
        ],
        out_shape=[
            jax.ShapeDtypeStruct((NPK, 128), jnp.float32),
            jax.ShapeDtypeStruct((NPK, 128), jnp.float32),
        ],
    )(hp, dcnt_p)


def _tc2(agg1_p, t1_p, dis_p, b1t):
    return pl.pallas_call(
        _tc2_body,
        grid=(1,),
        in_specs=[
            pl.BlockSpec((NC, NPK, 128), lambda i: (0, 0, 0)),
            pl.BlockSpec((NPK, 128), lambda i: (0, 0)),
            pl.BlockSpec((NPK, 128), lambda i: (0, 0)),
            pl.BlockSpec((1, 128), lambda i: (0, 0)),
        ],
        out_specs=pl.BlockSpec((NPK, 128), lambda i: (0, 0)),
        out_shape=jax.ShapeDtypeStruct((NPK, 128), jnp.float32),
    )(agg1_p, t1_p, dis_p, b1t)


def _tc3(agg2_p, t2_p, dis_p, W2, b2):
    return pl.pallas_call(
        _tc3_body,
        grid=(1,),
        in_specs=[
            pl.BlockSpec((NC, NPK, 128), lambda i: (0, 0, 0)),
            pl.BlockSpec((NPK, 128), lambda i: (0, 0)),
            pl.BlockSpec((NPK, 128), lambda i: (0, 0)),
            pl.BlockSpec((HID, OUT_CH), lambda i: (0, 0)),
            pl.BlockSpec((1, OUT_CH), lambda i: (0, 0)),
        ],
        out_specs=pl.BlockSpec((N_NODES, OUT_CH), lambda i: (0, 0)),
        out_shape=jax.ShapeDtypeStruct((N_NODES, OUT_CH), jnp.float32),
    )(agg2_p, t2_p, dis_p, W2, b2)


_sc_cache = {}


def _get_sc():
    if "agg1" not in _sc_cache:
        _sc_cache["agg1"] = _make_agg(HID, with_gather=True, kc0=128, kc1=32)
        _sc_cache["agg2"] = _make_agg(HID, with_gather=True, kc0=120, kc1=40)
        _sc_cache["deg"] = _make_agg(HID, with_gather=False)
    return _sc_cache["agg1"], _sc_cache["agg2"], _sc_cache["deg"]


@jax.jit
def kernel(x, edge_index, W1, b1, W2, b2):
    _agg1k, _agg2k, _deg16 = _get_sc()
    src = edge_index[0].astype(jnp.int32)
    dst = edge_index[1].astype(jnp.int32)
    def rho(n):
        q = jnp.floor((n.astype(jnp.float32) + 0.5) *
                      (1.0 / NPK)).astype(jnp.int32)
        return (n - q * NPK) * 8 + q
    src = rho(src)
    dst = rho(dst)
    pad = E_PAD - N_EDGES
    trash = (N_NODES % NPK) * 8 + N_NODES // NPK
    fill = jnp.full((pad,), trash, jnp.int32)
    src_p = jnp.concatenate([src, fill]).reshape(E_PAD // CHUNK, CHUNK)
    dst_p = jnp.concatenate([dst, fill]).reshape(E_PAD // CHUNK, CHUNK)

    hp = _tc1a(x, W1)
    dcnt = _deg16(src_p, dst_p)
    dcnt_p = dcnt.reshape(NC, NPK, 128)
    t1_p, dis_p = _tc1b(hp, dcnt_p)
    t1 = t1_p.reshape(N_PAD, HID)
    agg1 = _agg1k(t1, src_p, dst_p)
    t2_p = _tc2(agg1.reshape(NC, NPK, 128), t1_p, dis_p,
                jnp.tile(b1, 8).reshape(1, 128))
    t2 = t2_p.reshape(N_PAD, HID)
    agg2 = _agg2k(t2, src_p, dst_p)
    return _tc3(agg2.reshape(NC, NPK, 128), t2_p, dis_p, W2,
                b2.reshape(1, OUT_CH))


# speedup vs baseline: 2.4699x; 1.5354x over previous
import jax
import jax.numpy as jnp
from jax import lax
from jax.experimental import pallas as pl
from jax.experimental.pallas import tpu as pltpu
from jax.experimental.pallas import tpu_sc as plsc

N_NODES = 10000
IN_CH = 128
HID = 16
OUT_CH = 128
N_EDGES = 320000

NC = 2
NS = 16
CHUNK = 128
N_PAD = 10240
EPT = 10240
E_PAD = NC * NS * EPT
K_CHUNKS = EPT // CHUNK
KC0 = 112
KC1 = 48
RPT = N_PAD // NS
NKP = N_NODES // 8
NPK = N_PAD // 8


def _make_agg(feat, with_gather, kc0=KC0, kc1=KC1):
    mesh = plsc.VectorSubcoreMesh(core_axis_name="c", subcore_axis_name="s")
    NB = 4
    KD = 16
    scratch = [
        pltpu.VMEM((kc0, CHUNK), jnp.int32),
        pltpu.VMEM((kc0, CHUNK), jnp.int32),
        pltpu.VMEM((NB, CHUNK, feat), jnp.float32),
        pltpu.VMEM_SHARED((N_PAD, feat), jnp.float32),
        pltpu.VMEM_SHARED((N_PAD, feat), jnp.float32),
        pltpu.SemaphoreType.DMA((NB,)),
    ]

    def body(*refs):
        if with_gather:
            (t_hbm, src_hbm, dst_hbm, out_hbm,
             src_v, dst_v, rows_v, acc_sh, t_sh, sems) = refs
        else:
            (src_hbm, dst_hbm, out_hbm,
             src_v, dst_v, rows_v, acc_sh, t_sh, sems) = refs
        c = lax.axis_index("c")
        s = lax.axis_index("s")
        base = jnp.where(c == 0, s * kc0, NS * kc0 + s * kc1)
        nk = jnp.where(c == 0, kc0, kc1)

        @pl.when(c == 0)
        def _():
            pltpu.sync_copy(src_hbm.at[pl.ds(base, kc0)],
                            src_v.at[pl.ds(0, kc0)])
            pltpu.sync_copy(dst_hbm.at[pl.ds(base, kc0)],
                            dst_v.at[pl.ds(0, kc0)])

        @pl.when(c == 1)
        def _():
            pltpu.sync_copy(src_hbm.at[pl.ds(base, kc1)],
                            src_v.at[pl.ds(0, kc1)])
            pltpu.sync_copy(dst_hbm.at[pl.ds(base, kc1)],
                            dst_v.at[pl.ds(0, kc1)])

        def zstore(i, carry):
            rows_v[0, i, :] = jnp.zeros((feat,), jnp.float32)
            return carry
        lax.fori_loop(0, CHUNK, zstore, 0)
        for b in range(RPT // CHUNK):
            pltpu.sync_copy(
                rows_v.at[0], acc_sh.at[pl.ds(s * RPT + b * CHUNK, CHUNK)])

        if with_gather:
            pltpu.sync_copy(t_hbm.at[pl.ds(s * RPT, RPT)],
                            t_sh.at[pl.ds(s * RPT, RPT)])

        if not with_gather:
            def ostore(i, carry):
                rows_v[0, i, :] = jnp.ones((feat,), jnp.float32)
                return carry
            lax.fori_loop(0, CHUNK, ostore, 0)

        plsc.subcore_barrier()

        if with_gather:
            for b in range(NB):
                pltpu.async_copy(
                    t_sh.at[src_v.at[b]], rows_v.at[b], sems.at[b])

            def group(g, carry):
                for b in range(NB):
                    j = g * NB + b
                    pltpu.make_async_copy(
                        t_sh.at[src_v.at[0]], rows_v.at[b],
                        sems.at[b]).wait()
                    pltpu.sync_copy(
                        rows_v.at[b], acc_sh.at[dst_v.at[j]], add=True)
                    nxt = j + NB

                    @pl.when(nxt < nk)
                    def _():
                        pltpu.async_copy(
                            t_sh.at[src_v.at[nxt]], rows_v.at[b],
                            sems.at[b])
                return carry
            lax.fori_loop(0, nk // NB, group, 0)
        else:
            def dgroup(g, carry):
                def fire(j, carry2):
                    pltpu.async_copy(
                        rows_v.at[0], acc_sh.at[dst_v.at[g * KD + j]],
                        sems.at[0], add=True)
                    return carry2
                lax.fori_loop(0, KD, fire, 0)

                def drain(j, carry2):
                    pltpu.make_async_copy(
                        rows_v.at[0], acc_sh.at[dst_v.at[0]],
                        sems.at[0]).wait()
                    return carry2
                lax.fori_loop(0, KD, drain, 0)
                return carry
            lax.fori_loop(0, nk // KD, dgroup, 0)

        plsc.subcore_barrier()
        pltpu.sync_copy(acc_sh.at[pl.ds(s * RPT, RPT)],
                        out_hbm.at[c, pl.ds(s * RPT, RPT)])

    return pl.kernel(
        body,
        out_type=jax.ShapeDtypeStruct((NC, N_PAD, feat), jnp.float32),
        mesh=mesh,
        scratch_types=scratch,
        compiler_params=pltpu.CompilerParams(use_tc_tiling_on_sc=False),
    )


def _tc1a_body(x_ref, w1_ref, hp_ref):
    h = jnp.dot(x_ref[...], w1_ref[...],
                preferred_element_type=jnp.float32,
                precision=lax.Precision.HIGHEST)
    hpad = jnp.concatenate(
        [h, jnp.zeros((N_PAD - N_NODES, HID), jnp.float32)], 0)
    hp_ref[...] = jnp.concatenate(
        [hpad[i * NPK:(i + 1) * NPK] for i in range(8)], axis=1)


def _tc1b_body(hp_ref, dcnt_ref, t1_ref, dis_ref):
    deg = dcnt_ref[0] + dcnt_ref[1] + 1.0
    dis = lax.rsqrt(deg)
    dis_ref[...] = dis
    t1_ref[...] = hp_ref[...] * dis


def _tc2_body(agg_ref, t1_ref, dis_ref, b1_ref, t2_ref):
    dis = dis_ref[...]
    ssum = agg_ref[0] + agg_ref[1] + t1_ref[...]
    out1 = jnp.maximum(dis * ssum + b1_ref[...], 0.0)
    t2_ref[...] = out1 * dis


def _tc3_body(agg_ref, t2_ref, dis_ref, w2_ref, b2_ref, o_ref):
    midp = dis_ref[...] * (agg_ref[0] + agg_ref[1] + t2_ref[...])
    mid = jnp.concatenate(
        [midp[:, HID * i:HID * (i + 1)] for i in range(8)], axis=0)[:N_NODES]
    o = jnp.dot(mid, w2_ref[...],
                preferred_element_type=jnp.float32,
                precision=lax.Precision.HIGHEST) + b2_ref[...]
    m = jnp.max(o, axis=1, keepdims=True)
    lse = jnp.log(jnp.sum(jnp.exp(o - m), axis=1, keepdims=True))
    o_ref[...] = o - m - lse


def _tc1a(x, W1):
    return pl.pallas_call(
        _tc1a_body,
        grid=(1,),
        in_specs=[
            pl.BlockSpec((N_NODES, IN_CH), lambda i: (0, 0)),
            pl.BlockSpec((IN_CH, HID), lambda i: (0, 0)),
        ],
        out_specs=pl.BlockSpec((NPK, 128), lambda i: (0, 0)),
        out_shape=jax.ShapeDtypeStruct((NPK, 128), jnp.float32),
    )(x, W1)


def _tc1b(hp, dcnt_p):
    return pl.pallas_call(
        _tc1b_body,
        grid=(1,),
        in_specs=[
            pl.BlockSpec((NPK, 128), lambda i: (0, 0)),
            pl.BlockSpec((NC, NPK, 128), lambda i: (0, 0, 0)),
        ],
        out_specs=[
            pl.BlockSpec((NPK, 128), lambda i: (0, 0)),
            pl.BlockSpec((NPK, 128), lambda i: (0, 0)),
        ],
        out_shape=[
            jax.ShapeDtypeStruct((NPK, 128), jnp.float32),
            jax.ShapeDtypeStruct((NPK, 128), jnp.float32),
        ],
    )(hp, dcnt_p)


def _tc2(agg1_p, t1_p, dis_p, b1t):
    return pl.pallas_call(
        _tc2_body,
        grid=(1,),
        in_specs=[
            pl.BlockSpec((NC, NPK, 128), lambda i: (0, 0, 0)),
            pl.BlockSpec((NPK, 128), lambda i: (0, 0)),
            pl.BlockSpec((NPK, 128), lambda i: (0, 0)),
            pl.BlockSpec((1, 128), lambda i: (0, 0)),
        ],
        out_specs=pl.BlockSpec((NPK, 128), lambda i: (0, 0)),
        out_shape=jax.ShapeDtypeStruct((NPK, 128), jnp.float32),
    )(agg1_p, t1_p, dis_p, b1t)


def _tc3(agg2_p, t2_p, dis_p, W2, b2):
    return pl.pallas_call(
        _tc3_body,
        grid=(1,),
        in_specs=[
            pl.BlockSpec((NC, NPK, 128), lambda i: (0, 0, 0)),
            pl.BlockSpec((NPK, 128), lambda i: (0, 0)),
            pl.BlockSpec((NPK, 128), lambda i: (0, 0)),
            pl.BlockSpec((HID, OUT_CH), lambda i: (0, 0)),
            pl.BlockSpec((1, OUT_CH), lambda i: (0, 0)),
        ],
        out_specs=pl.BlockSpec((N_NODES, OUT_CH), lambda i: (0, 0)),
        out_shape=jax.ShapeDtypeStruct((N_NODES, OUT_CH), jnp.float32),
    )(agg2_p, t2_p, dis_p, W2, b2)


_sc_cache = {}


def _get_sc():
    if "agg1" not in _sc_cache:
        _sc_cache["agg1"] = _make_agg(HID, with_gather=True, kc0=80, kc1=80)
        _sc_cache["agg2"] = _make_agg(HID, with_gather=True, kc0=88, kc1=72)
        _sc_cache["deg"] = _make_agg(HID, with_gather=False)
    return _sc_cache["agg1"], _sc_cache["agg2"], _sc_cache["deg"]


@jax.jit
def kernel(x, edge_index, W1, b1, W2, b2):
    _agg1k, _agg2k, _deg16 = _get_sc()
    src = edge_index[0].astype(jnp.int32)
    dst = edge_index[1].astype(jnp.int32)
    def rho(n):
        q = jnp.floor((n.astype(jnp.float32) + 0.5) *
                      (1.0 / NPK)).astype(jnp.int32)
        return (n - q * NPK) * 8 + q
    src = rho(src)
    dst = rho(dst)
    pad = E_PAD - N_EDGES
    trash = (N_NODES % NPK) * 8 + N_NODES // NPK
    fill = jnp.full((pad,), trash, jnp.int32)
    src_p = jnp.concatenate([src, fill]).reshape(E_PAD // CHUNK, CHUNK)
    dst_p = jnp.concatenate([dst, fill]).reshape(E_PAD // CHUNK, CHUNK)

    hp = _tc1a(x, W1)
    dcnt = _deg16(src_p, dst_p)
    dcnt_p = dcnt.reshape(NC, NPK, 128)
    t1_p, dis_p = _tc1b(hp, dcnt_p)
    t1 = t1_p.reshape(N_PAD, HID)
    agg1 = _agg1k(t1, src_p, dst_p)
    t2_p = _tc2(agg1.reshape(NC, NPK, 128), t1_p, dis_p,
                jnp.tile(b1, 8).reshape(1, 128))
    t2 = t2_p.reshape(N_PAD, HID)
    agg2 = _agg2k(t2, src_p, dst_p)
    return _tc3(agg2.reshape(NC, NPK, 128), t2_p, dis_p, W2,
                b2.reshape(1, OUT_CH))


# speedup vs baseline: 2.7694x; 1.1212x over previous
import jax
import jax.numpy as jnp
from jax import lax
from jax.experimental import pallas as pl
from jax.experimental.pallas import tpu as pltpu
from jax.experimental.pallas import tpu_sc as plsc

N_NODES = 10000
IN_CH = 128
HID = 16
OUT_CH = 128
N_EDGES = 320000

NC = 2
NS = 16
CHUNK = 128
N_PAD = 10240
EPT = 10240
E_PAD = NC * NS * EPT
K_CHUNKS = EPT // CHUNK
KC0 = 112
KC1 = 48
RPT = N_PAD // NS
NKP = N_NODES // 8
NPK = N_PAD // 8


def _make_agg(feat, with_gather, kc0=KC0, kc1=KC1):
    mesh = plsc.VectorSubcoreMesh(core_axis_name="c", subcore_axis_name="s")
    NB = 4
    KD = 40
    scratch = [
        pltpu.VMEM((kc0, CHUNK), jnp.int32),
        pltpu.VMEM((kc0, CHUNK), jnp.int32),
        pltpu.VMEM((NB, CHUNK, feat), jnp.float32),
        pltpu.VMEM_SHARED((N_PAD, feat), jnp.float32),
        pltpu.VMEM_SHARED((N_PAD, feat), jnp.float32),
        pltpu.SemaphoreType.DMA((NB,)),
    ]

    def body(*refs):
        if with_gather:
            (t_hbm, e_hbm, out_hbm,
             src_v, dst_v, rows_v, acc_sh, t_sh, sems) = refs
        else:
            (e_hbm, out_hbm,
             src_v, dst_v, rows_v, acc_sh, t_sh, sems) = refs
        c = lax.axis_index("c")
        s = lax.axis_index("s")
        base = jnp.where(c == 0, s * kc0, NS * kc0 + s * kc1)
        nk = jnp.where(c == 0, kc0, kc1)

        @pl.when(c == 0)
        def _():
            pltpu.sync_copy(e_hbm.at[0, pl.ds(base, kc0)],
                            src_v.at[pl.ds(0, kc0)])
            pltpu.sync_copy(e_hbm.at[1, pl.ds(base, kc0)],
                            dst_v.at[pl.ds(0, kc0)])

        @pl.when(c == 1)
        def _():
            pltpu.sync_copy(e_hbm.at[0, pl.ds(base, kc1)],
                            src_v.at[pl.ds(0, kc1)])
            pltpu.sync_copy(e_hbm.at[1, pl.ds(base, kc1)],
                            dst_v.at[pl.ds(0, kc1)])

        def zstore(i, carry):
            rows_v[0, i, :] = jnp.zeros((feat,), jnp.float32)
            return carry
        lax.fori_loop(0, CHUNK, zstore, 0)
        for b in range(RPT // CHUNK):
            pltpu.sync_copy(
                rows_v.at[0], acc_sh.at[pl.ds(s * RPT + b * CHUNK, CHUNK)])

        if with_gather:
            pltpu.sync_copy(t_hbm.at[pl.ds(s * RPT, RPT)],
                            t_sh.at[pl.ds(s * RPT, RPT)])

        if not with_gather:
            def ostore(i, carry):
                rows_v[0, i, :] = jnp.ones((feat,), jnp.float32)
                return carry
            lax.fori_loop(0, CHUNK, ostore, 0)

        plsc.subcore_barrier()

        if with_gather:
            for b in range(NB):
                pltpu.async_copy(
                    t_sh.at[src_v.at[b]], rows_v.at[b], sems.at[b])

            def group(g, carry):
                for b in range(NB):
                    j = g * NB + b
                    pltpu.make_async_copy(
                        t_sh.at[src_v.at[0]], rows_v.at[b],
                        sems.at[b]).wait()
                    pltpu.sync_copy(
                        rows_v.at[b], acc_sh.at[dst_v.at[j]], add=True)
                    nxt = j + NB

                    @pl.when(nxt < nk)
                    def _():
                        pltpu.async_copy(
                            t_sh.at[src_v.at[nxt]], rows_v.at[b],
                            sems.at[b])
                return carry
            lax.fori_loop(0, nk // NB, group, 0)
        else:
            def dgroup(g, carry):
                def fire(j, carry2):
                    pltpu.async_copy(
                        rows_v.at[0], acc_sh.at[dst_v.at[g * KD + j]],
                        sems.at[0], add=True)
                    return carry2
                lax.fori_loop(0, KD, fire, 0)

                def drain(j, carry2):
                    pltpu.make_async_copy(
                        rows_v.at[0], acc_sh.at[dst_v.at[0]],
                        sems.at[0]).wait()
                    return carry2
                lax.fori_loop(0, KD, drain, 0)
                return carry
            lax.fori_loop(0, nk // KD, dgroup, 0)

        plsc.subcore_barrier()
        pltpu.sync_copy(acc_sh.at[pl.ds(s * RPT, RPT)],
                        out_hbm.at[c, pl.ds(s * RPT, RPT)])

    return pl.kernel(
        body,
        out_type=jax.ShapeDtypeStruct((NC, N_PAD, feat), jnp.float32),
        mesh=mesh,
        scratch_types=scratch,
        compiler_params=pltpu.CompilerParams(use_tc_tiling_on_sc=False),
    )


def _tc1a_body(x_ref, w1_ref, hp_ref):
    h = jnp.dot(x_ref[...], w1_ref[...],
                preferred_element_type=jnp.float32,
                precision=lax.Precision.HIGHEST)
    hpad = jnp.concatenate(
        [h, jnp.zeros((N_PAD - N_NODES, HID), jnp.float32)], 0)
    hp_ref[...] = jnp.concatenate(
        [hpad[i * NPK:(i + 1) * NPK] for i in range(8)], axis=1)


def _tc1b_body(hp_ref, dcnt_ref, t1_ref, dis_ref):
    deg = dcnt_ref[0] + dcnt_ref[1] + 1.0
    dis = lax.rsqrt(deg)
    dis_ref[...] = dis
    t1_ref[...] = hp_ref[...] * dis


def _tc2_body(agg_ref, t1_ref, dis_ref, b1_ref, t2_ref):
    dis = dis_ref[...]
    ssum = agg_ref[0] + agg_ref[1] + t1_ref[...]
    out1 = jnp.maximum(dis * ssum + b1_ref[...], 0.0)
    t2_ref[...] = out1 * dis


def _tc3_body(agg_ref, t2_ref, dis_ref, w2_ref, b2_ref, o_ref):
    midp = dis_ref[...] * (agg_ref[0] + agg_ref[1] + t2_ref[...])
    mid = jnp.concatenate(
        [midp[:, HID * i:HID * (i + 1)] for i in range(8)], axis=0)[:N_NODES]
    o = jnp.dot(mid, w2_ref[...],
                preferred_element_type=jnp.float32,
                precision=lax.Precision.HIGHEST) + b2_ref[...]
    m = jnp.max(o, axis=1, keepdims=True)
    lse = jnp.log(jnp.sum(jnp.exp(o - m), axis=1, keepdims=True))
    o_ref[...] = o - m - lse


def _tc1a(x, W1):
    return pl.pallas_call(
        _tc1a_body,
        grid=(1,),
        in_specs=[
            pl.BlockSpec((N_NODES, IN_CH), lambda i: (0, 0)),
            pl.BlockSpec((IN_CH, HID), lambda i: (0, 0)),
        ],
        out_specs=pl.BlockSpec((NPK, 128), lambda i: (0, 0)),
        out_shape=jax.ShapeDtypeStruct((NPK, 128), jnp.float32),
    )(x, W1)


def _tc1b(hp, dcnt_p):
    return pl.pallas_call(
        _tc1b_body,
        grid=(1,),
        in_specs=[
            pl.BlockSpec((NPK, 128), lambda i: (0, 0)),
            pl.BlockSpec((NC, NPK, 128), lambda i: (0, 0, 0)),
        ],
        out_specs=[
            pl.BlockSpec((NPK, 128), lambda i: (0, 0)),
            pl.BlockSpec((NPK, 128), lambda i: (0, 0)),
        ],
        out_shape=[
            jax.ShapeDtypeStruct((NPK, 128), jnp.float32),
            jax.ShapeDtypeStruct((NPK, 128), jnp.float32),
        ],
    )(hp, dcnt_p)


def _tc2(agg1_p, t1_p, dis_p, b1t):
    return pl.pallas_call(
        _tc2_body,
        grid=(1,),
        in_specs=[
            pl.BlockSpec((NC, NPK, 128), lambda i: (0, 0, 0)),
            pl.BlockSpec((NPK, 128), lambda i: (0, 0)),
            pl.BlockSpec((NPK, 128), lambda i: (0, 0)),
            pl.BlockSpec((1, 128), lambda i: (0, 0)),
        ],
        out_specs=pl.BlockSpec((NPK, 128), lambda i: (0, 0)),
        out_shape=jax.ShapeDtypeStruct((NPK, 128), jnp.float32),
    )(agg1_p, t1_p, dis_p, b1t)


def _tc3(agg2_p, t2_p, dis_p, W2, b2):
    return pl.pallas_call(
        _tc3_body,
        grid=(1,),
        in_specs=[
            pl.BlockSpec((NC, NPK, 128), lambda i: (0, 0, 0)),
            pl.BlockSpec((NPK, 128), lambda i: (0, 0)),
            pl.BlockSpec((NPK, 128), lambda i: (0, 0)),
            pl.BlockSpec((HID, OUT_CH), lambda i: (0, 0)),
            pl.BlockSpec((1, OUT_CH), lambda i: (0, 0)),
        ],
        out_specs=pl.BlockSpec((N_NODES, OUT_CH), lambda i: (0, 0)),
        out_shape=jax.ShapeDtypeStruct((N_NODES, OUT_CH), jnp.float32),
    )(agg2_p, t2_p, dis_p, W2, b2)


_sc_cache = {}


def _get_sc():
    if "agg1" not in _sc_cache:
        _sc_cache["agg1"] = _make_agg(HID, with_gather=True, kc0=80, kc1=80)
        _sc_cache["agg2"] = _make_agg(HID, with_gather=True, kc0=88, kc1=72)
        _sc_cache["deg"] = _make_agg(HID, with_gather=False)
    return _sc_cache["agg1"], _sc_cache["agg2"], _sc_cache["deg"]


@jax.jit
def kernel(x, edge_index, W1, b1, W2, b2):
    _agg1k, _agg2k, _deg16 = _get_sc()
    e = edge_index.astype(jnp.int32)
    e = jnp.pad(e, ((0, 0), (0, E_PAD - N_EDGES)), constant_values=N_NODES)

    def rho(n):
        q = jnp.floor((n.astype(jnp.float32) + 0.5) *
                      (1.0 / NPK)).astype(jnp.int32)
        return (n - q * NPK) * 8 + q
    e_p = rho(e).reshape(2, E_PAD // CHUNK, CHUNK)

    hp = _tc1a(x, W1)
    dcnt = _deg16(e_p)
    dcnt_p = dcnt.reshape(NC, NPK, 128)
    t1_p, dis_p = _tc1b(hp, dcnt_p)
    t1 = t1_p.reshape(N_PAD, HID)
    agg1 = _agg1k(t1, e_p)
    t2_p = _tc2(agg1.reshape(NC, NPK, 128), t1_p, dis_p,
                jnp.tile(b1, 8).reshape(1, 128))
    t2 = t2_p.reshape(N_PAD, HID)
    agg2 = _agg2k(t2, e_p)
    return _tc3(agg2.reshape(NC, NPK, 128), t2_p, dis_p, W2,
                b2.reshape(1, OUT_CH))
